# Initial kernel scaffold; baseline (speedup 1.0000x reference)
import jax
import jax.numpy as jnp
from jax.experimental import pallas as pl


def kernel(x, local_indices, group_mask, pe):
    raise NotImplementedError("write your pallas kernel here")


# speedup vs baseline: 2.9650x; 2.9650x over previous
import functools

import jax
import jax.numpy as jnp
from jax import lax
from jax.experimental import pallas as pl
from jax.experimental.pallas import tpu as pltpu


def _tc_body(g, x_ref, idx_ref, mask_ref, pe_ref, out_ref):
    i = pl.program_id(0)
    idxb = idx_ref[i, 0, :]
    maskb = mask_ref[i, 0, :]
    iota = lax.broadcasted_iota(jnp.int32, (idxb.shape[0], g), 1)
    onehot = ((idxb[:, None] == iota) & (maskb[:, None] != 0)).astype(jnp.float32)
    out_ref[...] = x_ref[...] + jnp.dot(
        onehot, pe_ref[...], preferred_element_type=jnp.float32
    )


def kernel(x, local_indices, group_mask, pe):
    b, s, d = x.shape
    g = pe.shape[0]
    n = b * s
    t = 1024
    nb = n // t

    xf = x.reshape(n, d)
    idx = local_indices.reshape(nb, 1, t).astype(jnp.int32)
    mask = group_mask.reshape(nb, 1, t).astype(jnp.int32)

    out = pl.pallas_call(
        functools.partial(_tc_body, g),
        grid=(nb,),
        in_specs=[
            pl.BlockSpec((t, d), lambda i: (i, 0)),
            pl.BlockSpec((nb, 1, t), lambda i: (0, 0, 0)),
            pl.BlockSpec((nb, 1, t), lambda i: (0, 0, 0)),
            pl.BlockSpec((g, d), lambda i: (0, 0)),
        ],
        out_specs=pl.BlockSpec((t, d), lambda i: (i, 0)),
        out_shape=jax.ShapeDtypeStruct((n, d), x.dtype),
        compiler_params=pltpu.CompilerParams(
            dimension_semantics=("arbitrary",),
        ),
    )(xf, idx, mask, pe)
    return out.reshape(b, s, d)
